# baseline (speedup 1.0000x reference)
import functools

import jax
import jax.numpy as jnp
from jax.experimental import pallas as pl
from jax.experimental.pallas import tpu as pltpu
from jax.experimental.pallas import tpu_sc as plsc

_QB = 512
_N_QUERY = 1024
_N_TRAIN = 16384
_D = 16


def _nn_block_kernel(x_ref, yt_ref, out_ref):
    x = x_ref[...]
    yt = yt_ref[...]
    ynorm = jnp.sum(yt * yt, axis=0, keepdims=True)
    f32 = jnp.float32
    bf16 = jnp.bfloat16
    m2x = -2.0 * x
    m2x_hi = m2x.astype(bf16)
    m2x_lo = (m2x - m2x_hi.astype(f32)).astype(bf16)
    yt_hi = yt.astype(bf16)
    yt_lo = (yt - yt_hi.astype(f32)).astype(bf16)
    yn_hi = ynorm.astype(bf16)
    yn_r = ynorm - yn_hi.astype(f32)
    yn_mid = yn_r.astype(bf16)
    yn_lo = (yn_r - yn_mid.astype(f32)).astype(bf16)
    ones = jnp.ones((_QB, 1), bf16)
    xk = jnp.concatenate(
        [m2x_hi, m2x_hi, m2x_lo, ones, ones, ones], axis=1)
    ytk = jnp.concatenate(
        [yt_hi, yt_lo, yt_hi, yn_hi, yn_mid, yn_lo], axis=0)
    dist = jax.lax.dot_general(
        xk, ytk, (((1,), (0,)), ((), ())),
        preferred_element_type=jnp.float32)
    first_idx = jnp.argmin(dist, axis=1)
    out_ref[...] = first_idx.reshape(1, 1, _QB)


_SC_INFO = plsc.get_sparse_core_info()
_NW = _SC_INFO.num_cores * _SC_INFO.num_subcores
_BPW = _N_QUERY // _NW


def _sc_gather(labels, idx):
    mesh = plsc.VectorSubcoreMesh(core_axis_name="c", subcore_axis_name="s")

    @functools.partial(
        pl.kernel, mesh=mesh,
        out_type=jax.ShapeDtypeStruct((_N_QUERY,), jnp.float32),
        scratch_types=[
            pltpu.VMEM((_BPW,), jnp.int32),
            pltpu.VMEM((_BPW,), jnp.float32),
            pltpu.SemaphoreType.DMA,
        ])
    def k(lab_hbm, idx_hbm, out_hbm, idx_v, vals_v, sem):
        wid = jax.lax.axis_index("s") * _SC_INFO.num_cores + jax.lax.axis_index("c")
        base = wid * _BPW
        pltpu.sync_copy(idx_hbm.at[pl.ds(base, _BPW)], idx_v)
        pltpu.async_copy(lab_hbm.at[idx_v], vals_v, sem).wait()
        pltpu.sync_copy(vals_v, out_hbm.at[pl.ds(base, _BPW)])

    return k(labels, idx)


def kernel(x, train_pts, train_label):
    yt = train_pts.T
    n_blocks = _N_QUERY // _QB
    idx = pl.pallas_call(
        _nn_block_kernel,
        grid=(n_blocks,),
        in_specs=[
            pl.BlockSpec((_QB, _D), lambda i: (i, 0)),
            pl.BlockSpec((_D, _N_TRAIN), lambda i: (0, 0)),
        ],
        out_specs=pl.BlockSpec((1, 1, _QB), lambda i: (i, 0, 0)),
        out_shape=jax.ShapeDtypeStruct((n_blocks, 1, _QB), jnp.int32),
    )(x, yt)
    return _sc_gather(train_label, idx.reshape(_N_QUERY))
